# Initial kernel scaffold; baseline (speedup 1.0000x reference)
import jax
import jax.numpy as jnp
from jax.experimental import pallas as pl


def kernel(x, edge_index, edge_attr, batch, Wf0, bf0, Ws0, bs0, g0, b0, Wf1, bf1, Ws1, bs1, g1, b1):
    raise NotImplementedError("write your pallas kernel here")


# speedup vs baseline: 2.0551x; 2.0551x over previous
import functools

import jax
import jax.numpy as jnp
from jax import lax
from jax.experimental import pallas as pl
from jax.experimental.pallas import tpu as pltpu
from jax.experimental.pallas import tpu_sc as plsc

N_NODES = 10000
N_EDGES = 320000
D = 128
D_EDGE = 16
N_GRAPHS = 64

NC = 2
NS = 16
NW = NC * NS
EDGES_PER_W = N_EDGES // NW
CHUNK = 80
NCHUNK = EDGES_PER_W // CHUNK
N_PAD = 10240
ROWS_PER_TILE = N_PAD // NS

F32 = jnp.float32


@functools.cache
def _sc_kernels():
    mesh = plsc.VectorSubcoreMesh(
        core_axis_name="c", subcore_axis_name="s",
        num_cores=NC, num_subcores=NS)

    @functools.partial(
        pl.kernel,
        out_type=[jax.ShapeDtypeStruct((N_EDGES, 2 * D), F32),
                  jax.ShapeDtypeStruct((N_EDGES, 2 * D), F32)],
        mesh=mesh,
        scratch_types=[pltpu.VMEM((CHUNK,), jnp.int32),
                       pltpu.VMEM((CHUNK,), jnp.int32),
                       pltpu.VMEM((CHUNK, 2 * D), F32),
                       pltpu.VMEM((CHUNK, 2 * D), F32),
                       pltpu.SemaphoreType.DMA,
                       pltpu.SemaphoreType.DMA],
    )
    def _sc_gather(pd_hbm, ps_hbm, src_hbm, dst_hbm, gd_out, gs_out,
                   idx_d, idx_s, rows_d, rows_s, sem_d, sem_s):
        c = lax.axis_index("c")
        s = lax.axis_index("s")
        w = c * NS + s

        def body(j, carry):
            base = w * EDGES_PER_W + j * CHUNK
            pltpu.sync_copy(dst_hbm.at[pl.ds(base, CHUNK)], idx_d)
            pltpu.sync_copy(src_hbm.at[pl.ds(base, CHUNK)], idx_s)
            cp_d = pltpu.async_copy(pd_hbm.at[idx_d], rows_d, sem_d)
            cp_s = pltpu.async_copy(ps_hbm.at[idx_s], rows_s, sem_s)
            cp_d.wait()
            cp_s.wait()
            pltpu.sync_copy(rows_d, gd_out.at[pl.ds(base, CHUNK)])
            pltpu.sync_copy(rows_s, gs_out.at[pl.ds(base, CHUNK)])
            return carry

        lax.fori_loop(0, NCHUNK, body, 0)

    @functools.partial(
        pl.kernel,
        out_type=[jax.ShapeDtypeStruct((N_PAD, D), F32),
                  jax.ShapeDtypeStruct((N_PAD, D), F32)],
        mesh=mesh,
        scratch_types=[pltpu.VMEM((CHUNK,), jnp.int32),
                       pltpu.VMEM((CHUNK, D), F32),
                       pltpu.MemorySpace.VMEM_SHARED((N_PAD, D), F32)],
    )
    def _sc_scatter(msg_hbm, dst_hbm, zeros_hbm, out_a, out_b,
                    idx_v, msg_v, agg_sh):
        c = lax.axis_index("c")
        s = lax.axis_index("s")
        row0 = s * ROWS_PER_TILE
        pltpu.sync_copy(zeros_hbm.at[pl.ds(row0, ROWS_PER_TILE)],
                        agg_sh.at[pl.ds(row0, ROWS_PER_TILE)])
        plsc.subcore_barrier()

        def body(j, carry):
            base = (c * NS + s) * EDGES_PER_W + j * CHUNK
            pltpu.sync_copy(dst_hbm.at[pl.ds(base, CHUNK)], idx_v)
            pltpu.sync_copy(msg_hbm.at[pl.ds(base, CHUNK)], msg_v)
            pltpu.sync_copy(msg_v, agg_sh.at[idx_v], add=True)
            return carry

        lax.fori_loop(0, NCHUNK, body, 0)
        plsc.subcore_barrier()

        @pl.when(c == 0)
        def _():
            pltpu.sync_copy(agg_sh.at[pl.ds(row0, ROWS_PER_TILE)],
                            out_a.at[pl.ds(row0, ROWS_PER_TILE)])

        @pl.when(c == 1)
        def _():
            pltpu.sync_copy(agg_sh.at[pl.ds(row0, ROWS_PER_TILE)],
                            out_b.at[pl.ds(row0, ROWS_PER_TILE)])

    return _sc_gather, _sc_scatter


_NODE_BLK = 1000
_N_NODE_BLKS = N_NODES // _NODE_BLK
_EDGE_BLK = 2000
_N_EDGE_BLKS = N_EDGES // _EDGE_BLK


def _norm_matmul_body(y_ref, ac_ref, w_ref, h_ref, pd_ref, ps_ref):
    a = ac_ref[0:1, :]
    cc = ac_ref[1:2, :]
    h = y_ref[...] * a + cc
    h_ref[...] = h
    p = jnp.dot(h, w_ref[...], preferred_element_type=F32)
    pd_ref[...] = p[:, :2 * D]
    ps_ref[...] = p[:, 2 * D:]


def _norm_matmul(y, ac, wc):
    return pl.pallas_call(
        _norm_matmul_body,
        grid=(_N_NODE_BLKS,),
        in_specs=[pl.BlockSpec((_NODE_BLK, D), lambda i: (i, 0)),
                  pl.BlockSpec((8, D), lambda i: (0, 0)),
                  pl.BlockSpec((D, 4 * D), lambda i: (0, 0))],
        out_specs=[pl.BlockSpec((_NODE_BLK, D), lambda i: (i, 0)),
                   pl.BlockSpec((_NODE_BLK, 2 * D), lambda i: (i, 0)),
                   pl.BlockSpec((_NODE_BLK, 2 * D), lambda i: (i, 0))],
        out_shape=[jax.ShapeDtypeStruct((N_NODES, D), F32),
                   jax.ShapeDtypeStruct((N_NODES, 2 * D), F32),
                   jax.ShapeDtypeStruct((N_NODES, 2 * D), F32)],
    )(y, ac, wc)


def _msg_body(gd_ref, gs_ref, ea_ref, we_ref, bias_ref, msg_ref):
    e2 = jnp.dot(ea_ref[...], we_ref[...], preferred_element_type=F32)
    e2 = e2 + bias_ref[0:1, :]
    zf = gd_ref[:, :D] + gs_ref[:, :D] + e2[:, :D]
    zs = gd_ref[:, D:] + gs_ref[:, D:] + e2[:, D:]
    msg_ref[...] = jax.nn.sigmoid(zf) * jax.nn.softplus(zs)


def _msg_kernel(gd, gs, ea, we, bias):
    return pl.pallas_call(
        _msg_body,
        grid=(_N_EDGE_BLKS,),
        in_specs=[pl.BlockSpec((_EDGE_BLK, 2 * D), lambda i: (i, 0)),
                  pl.BlockSpec((_EDGE_BLK, 2 * D), lambda i: (i, 0)),
                  pl.BlockSpec((_EDGE_BLK, D_EDGE), lambda i: (i, 0)),
                  pl.BlockSpec((D_EDGE, 2 * D), lambda i: (0, 0)),
                  pl.BlockSpec((8, 2 * D), lambda i: (0, 0))],
        out_specs=pl.BlockSpec((_EDGE_BLK, D), lambda i: (i, 0)),
        out_shape=jax.ShapeDtypeStruct((N_EDGES, D), F32),
    )(gd, gs, ea, we, bias)


def _stats_update(st_ref, y, i):
    s = jnp.sum(y, axis=0)[None, :]
    q = jnp.sum(y * y, axis=0)[None, :]
    upd = jnp.concatenate([s, q, jnp.zeros((6, D), F32)], axis=0)

    @pl.when(i == 0)
    def _():
        st_ref[...] = upd

    @pl.when(i > 0)
    def _():
        st_ref[...] = st_ref[...] + upd


def _add_stats_body(x_ref, aa_ref, ab_ref, y_ref, st_ref):
    i = pl.program_id(0)
    y = x_ref[...] + aa_ref[...] + ab_ref[...]
    y_ref[...] = y
    _stats_update(st_ref, y, i)


def _add_stats(x, agg_a, agg_b):
    return pl.pallas_call(
        _add_stats_body,
        grid=(_N_NODE_BLKS,),
        in_specs=[pl.BlockSpec((_NODE_BLK, D), lambda i: (i, 0))] * 3,
        out_specs=[pl.BlockSpec((_NODE_BLK, D), lambda i: (i, 0)),
                   pl.BlockSpec((8, D), lambda i: (0, 0))],
        out_shape=[jax.ShapeDtypeStruct((N_NODES, D), F32),
                   jax.ShapeDtypeStruct((8, D), F32)],
    )(x, agg_a, agg_b)


def _finalize_body(h_ref, aa_ref, ab_ref, b_ref, st_ref, pool_ref, cnt_ref):
    i = pl.program_id(0)
    y = h_ref[...] + aa_ref[...] + ab_ref[...]
    _stats_update(st_ref, y, i)
    lanes = lax.broadcasted_iota(jnp.int32, (_NODE_BLK, D), 1)
    oh = (b_ref[...] == lanes).astype(F32)
    pool_upd = lax.dot_general(oh, y, (((0,), (0,)), ((), ())),
                               preferred_element_type=F32)
    cnt_upd = jnp.concatenate(
        [jnp.sum(oh, axis=0)[None, :], jnp.zeros((7, D), F32)], axis=0)

    @pl.when(i == 0)
    def _():
        pool_ref[...] = pool_upd
        cnt_ref[...] = cnt_upd

    @pl.when(i > 0)
    def _():
        pool_ref[...] = pool_ref[...] + pool_upd
        cnt_ref[...] = cnt_ref[...] + cnt_upd


def _finalize(h, agg_a, agg_b, batch2d):
    return pl.pallas_call(
        _finalize_body,
        grid=(_N_NODE_BLKS,),
        in_specs=[pl.BlockSpec((_NODE_BLK, D), lambda i: (i, 0))] * 3 +
                 [pl.BlockSpec((_NODE_BLK, 1), lambda i: (i, 0))],
        out_specs=[pl.BlockSpec((8, D), lambda i: (0, 0)),
                   pl.BlockSpec((D, D), lambda i: (0, 0)),
                   pl.BlockSpec((8, D), lambda i: (0, 0))],
        out_shape=[jax.ShapeDtypeStruct((8, D), F32),
                   jax.ShapeDtypeStruct((D, D), F32),
                   jax.ShapeDtypeStruct((8, D), F32)],
    )(h, agg_a, agg_b, batch2d)


def _bn_affine(st, gamma, beta, eps=1e-5):
    mean = st[0] / N_NODES
    var = st[1] / N_NODES - mean * mean
    a = gamma * lax.rsqrt(var + eps)
    return a, beta - mean * a


def _pack_weights(wf, ws):
    wc = jnp.concatenate(
        [wf[:D], ws[:D], wf[D:2 * D], ws[D:2 * D]], axis=1)
    we = jnp.concatenate([wf[2 * D:], ws[2 * D:]], axis=1)
    return wc, we


def _pad8(row_list):
    rows = jnp.stack(row_list, axis=0)
    return jnp.concatenate(
        [rows, jnp.zeros((8 - rows.shape[0], rows.shape[1]), F32)], axis=0)


def kernel(x, edge_index, edge_attr, batch, Wf0, bf0, Ws0, bs0, g0, b0,
           Wf1, bf1, Ws1, bs1, g1, b1):
    src = edge_index[0]
    dst = edge_index[1]
    zeros_nd = jnp.zeros((N_PAD, D), F32)
    batch2d = batch.reshape(N_NODES, 1)

    wc0, we0 = _pack_weights(Wf0, Ws0)
    wc1, we1 = _pack_weights(Wf1, Ws1)
    bias0 = _pad8([jnp.concatenate([bf0, bs0])])
    bias1 = _pad8([jnp.concatenate([bf1, bs1])])
    ac_id = _pad8([jnp.ones((D,), F32), jnp.zeros((D,), F32)])
    sc_gather, sc_scatter = _sc_kernels()

    _, pd1, ps1 = _norm_matmul(x, ac_id, wc0)
    gd1, gs1 = sc_gather(pd1, ps1, src, dst)
    msg1 = _msg_kernel(gd1, gs1, edge_attr, we0, bias0)
    agg1a, agg1b = sc_scatter(msg1, dst, zeros_nd)
    y1, st1 = _add_stats(x, agg1a[:N_NODES], agg1b[:N_NODES])
    a1, c1 = _bn_affine(st1, g0, b0)

    h1, pd2, ps2 = _norm_matmul(y1, _pad8([a1, c1]), wc1)
    gd2, gs2 = sc_gather(pd2, ps2, src, dst)
    msg2 = _msg_kernel(gd2, gs2, edge_attr, we1, bias1)
    agg2a, agg2b = sc_scatter(msg2, dst, zeros_nd)
    st2, pooled, cnt = _finalize(h1, agg2a[:N_NODES], agg2b[:N_NODES], batch2d)

    a2, c2 = _bn_affine(st2, g1, b1)
    counts = jnp.maximum(cnt[0, :N_GRAPHS], 1.0)
    pool_mean = pooled[:N_GRAPHS] / counts[:, None]
    return pool_mean * a2[None, :] + c2[None, :]


# speedup vs baseline: 2.4011x; 1.1684x over previous
import functools

import jax
import jax.numpy as jnp
from jax import lax
from jax.experimental import pallas as pl
from jax.experimental.pallas import tpu as pltpu
from jax.experimental.pallas import tpu_sc as plsc

N_NODES = 10000
N_EDGES = 320000
D = 128
D_EDGE = 16
N_GRAPHS = 64

NC = 2
NS = 16
NW = NC * NS
EDGES_PER_W = N_EDGES // NW
CHUNK = 80
NCHUNK = EDGES_PER_W // CHUNK
N_PAD = 10240
ROWS_PER_TILE = N_PAD // NS

F32 = jnp.float32


@functools.cache
def _sc_kernels():
    mesh = plsc.VectorSubcoreMesh(
        core_axis_name="c", subcore_axis_name="s",
        num_cores=NC, num_subcores=NS)

    @functools.partial(
        pl.kernel,
        out_type=[jax.ShapeDtypeStruct((N_EDGES, 2 * D), F32),
                  jax.ShapeDtypeStruct((N_EDGES, 2 * D), F32)],
        mesh=mesh,
        scratch_types=[pltpu.VMEM((CHUNK,), jnp.int32),
                       pltpu.VMEM((CHUNK,), jnp.int32),
                       pltpu.VMEM((CHUNK,), jnp.int32),
                       pltpu.VMEM((CHUNK,), jnp.int32),
                       pltpu.VMEM((CHUNK, 2 * D), F32),
                       pltpu.VMEM((CHUNK, 2 * D), F32),
                       pltpu.VMEM((CHUNK, 2 * D), F32),
                       pltpu.VMEM((CHUNK, 2 * D), F32),
                       pltpu.SemaphoreType.DMA,
                       pltpu.SemaphoreType.DMA,
                       pltpu.SemaphoreType.DMA,
                       pltpu.SemaphoreType.DMA],
    )
    def _sc_gather(pd_hbm, ps_hbm, src_hbm, dst_hbm, gd_out, gs_out,
                   ida_d, ida_s, idb_d, idb_s,
                   rowsa_d, rowsa_s, rowsb_d, rowsb_s,
                   sema_d, sema_s, semb_d, semb_s):
        c = lax.axis_index("c")
        s = lax.axis_index("s")
        w = c * NS + s
        first = w * EDGES_PER_W

        def load_idx(base, id_d, id_s):
            pltpu.sync_copy(dst_hbm.at[pl.ds(base, CHUNK)], id_d)
            pltpu.sync_copy(src_hbm.at[pl.ds(base, CHUNK)], id_s)

        def start(id_d, id_s, r_d, r_s, se_d, se_s):
            pltpu.async_copy(pd_hbm.at[id_d], r_d, se_d)
            pltpu.async_copy(ps_hbm.at[id_s], r_s, se_s)

        def drain(id_d, id_s, r_d, r_s, se_d, se_s):
            pltpu.make_async_copy(pd_hbm.at[id_d], r_d, se_d).wait()
            pltpu.make_async_copy(ps_hbm.at[id_s], r_s, se_s).wait()

        def store(base, r_d, r_s):
            pltpu.sync_copy(r_d, gd_out.at[pl.ds(base, CHUNK)])
            pltpu.sync_copy(r_s, gs_out.at[pl.ds(base, CHUNK)])

        load_idx(first, ida_d, ida_s)
        start(ida_d, ida_s, rowsa_d, rowsa_s, sema_d, sema_s)

        def body(i, carry):
            base_a = first + 2 * i * CHUNK
            base_b = base_a + CHUNK
            load_idx(base_b, idb_d, idb_s)
            start(idb_d, idb_s, rowsb_d, rowsb_s, semb_d, semb_s)
            drain(ida_d, ida_s, rowsa_d, rowsa_s, sema_d, sema_s)
            store(base_a, rowsa_d, rowsa_s)
            load_idx(base_b + CHUNK, ida_d, ida_s)
            start(ida_d, ida_s, rowsa_d, rowsa_s, sema_d, sema_s)
            drain(idb_d, idb_s, rowsb_d, rowsb_s, semb_d, semb_s)
            store(base_b, rowsb_d, rowsb_s)
            return carry

        lax.fori_loop(0, (NCHUNK - 1) // 2, body, 0)
        drain(ida_d, ida_s, rowsa_d, rowsa_s, sema_d, sema_s)
        store(first + (NCHUNK - 1) * CHUNK, rowsa_d, rowsa_s)

    @functools.partial(
        pl.kernel,
        out_type=[jax.ShapeDtypeStruct((N_PAD, D), F32),
                  jax.ShapeDtypeStruct((N_PAD, D), F32)],
        mesh=mesh,
        scratch_types=[pltpu.VMEM((CHUNK,), jnp.int32),
                       pltpu.VMEM((CHUNK,), jnp.int32),
                       pltpu.VMEM((CHUNK, D), F32),
                       pltpu.VMEM((CHUNK, D), F32),
                       pltpu.MemorySpace.VMEM_SHARED((N_PAD, D), F32),
                       pltpu.SemaphoreType.DMA,
                       pltpu.SemaphoreType.DMA],
    )
    def _sc_scatter(msg_hbm, dst_hbm, zeros_hbm, out_a, out_b,
                    idxa, idxb, msga, msgb, agg_sh, sema, semb):
        c = lax.axis_index("c")
        s = lax.axis_index("s")
        row0 = s * ROWS_PER_TILE
        first = (c * NS + s) * EDGES_PER_W
        pltpu.sync_copy(zeros_hbm.at[pl.ds(row0, ROWS_PER_TILE)],
                        agg_sh.at[pl.ds(row0, ROWS_PER_TILE)])
        plsc.subcore_barrier()

        def load(base, idx_v, msg_v):
            pltpu.sync_copy(dst_hbm.at[pl.ds(base, CHUNK)], idx_v)
            pltpu.sync_copy(msg_hbm.at[pl.ds(base, CHUNK)], msg_v)

        load(first, idxa, msga)

        def body(i, carry):
            base_b = first + (2 * i + 1) * CHUNK
            pltpu.async_copy(msga, agg_sh.at[idxa], sema, add=True)
            load(base_b, idxb, msgb)
            pltpu.make_async_copy(msga, agg_sh.at[idxa], sema).wait()
            pltpu.async_copy(msgb, agg_sh.at[idxb], semb, add=True)
            load(base_b + CHUNK, idxa, msga)
            pltpu.make_async_copy(msgb, agg_sh.at[idxb], semb).wait()
            return carry

        lax.fori_loop(0, (NCHUNK - 1) // 2, body, 0)
        pltpu.async_copy(msga, agg_sh.at[idxa], sema, add=True)
        pltpu.make_async_copy(msga, agg_sh.at[idxa], sema).wait()
        plsc.subcore_barrier()

        @pl.when(c == 0)
        def _():
            pltpu.sync_copy(agg_sh.at[pl.ds(row0, ROWS_PER_TILE)],
                            out_a.at[pl.ds(row0, ROWS_PER_TILE)])

        @pl.when(c == 1)
        def _():
            pltpu.sync_copy(agg_sh.at[pl.ds(row0, ROWS_PER_TILE)],
                            out_b.at[pl.ds(row0, ROWS_PER_TILE)])

    return _sc_gather, _sc_scatter


_NODE_BLK = 1000
_N_NODE_BLKS = N_NODES // _NODE_BLK
_EDGE_BLK = 2000
_N_EDGE_BLKS = N_EDGES // _EDGE_BLK


def _norm_matmul_body(y_ref, ac_ref, w_ref, h_ref, pd_ref, ps_ref):
    a = ac_ref[0:1, :]
    cc = ac_ref[1:2, :]
    h = y_ref[...] * a + cc
    h_ref[...] = h
    p = jnp.dot(h, w_ref[...], preferred_element_type=F32)
    pd_ref[...] = p[:, :2 * D]
    ps_ref[...] = p[:, 2 * D:]


def _norm_matmul(y, ac, wc):
    return pl.pallas_call(
        _norm_matmul_body,
        grid=(_N_NODE_BLKS,),
        in_specs=[pl.BlockSpec((_NODE_BLK, D), lambda i: (i, 0)),
                  pl.BlockSpec((8, D), lambda i: (0, 0)),
                  pl.BlockSpec((D, 4 * D), lambda i: (0, 0))],
        out_specs=[pl.BlockSpec((_NODE_BLK, D), lambda i: (i, 0)),
                   pl.BlockSpec((_NODE_BLK, 2 * D), lambda i: (i, 0)),
                   pl.BlockSpec((_NODE_BLK, 2 * D), lambda i: (i, 0))],
        out_shape=[jax.ShapeDtypeStruct((N_NODES, D), F32),
                   jax.ShapeDtypeStruct((N_NODES, 2 * D), F32),
                   jax.ShapeDtypeStruct((N_NODES, 2 * D), F32)],
    )(y, ac, wc)


def _msg_body(gd_ref, gs_ref, ea_ref, we_ref, bias_ref, msg_ref):
    e2 = jnp.dot(ea_ref[...], we_ref[...], preferred_element_type=F32)
    e2 = e2 + bias_ref[0:1, :]
    zf = gd_ref[:, :D] + gs_ref[:, :D] + e2[:, :D]
    zs = gd_ref[:, D:] + gs_ref[:, D:] + e2[:, D:]
    msg_ref[...] = jax.nn.sigmoid(zf) * jax.nn.softplus(zs)


def _msg_kernel(gd, gs, ea, we, bias):
    return pl.pallas_call(
        _msg_body,
        grid=(_N_EDGE_BLKS,),
        in_specs=[pl.BlockSpec((_EDGE_BLK, 2 * D), lambda i: (i, 0)),
                  pl.BlockSpec((_EDGE_BLK, 2 * D), lambda i: (i, 0)),
                  pl.BlockSpec((_EDGE_BLK, D_EDGE), lambda i: (i, 0)),
                  pl.BlockSpec((D_EDGE, 2 * D), lambda i: (0, 0)),
                  pl.BlockSpec((8, 2 * D), lambda i: (0, 0))],
        out_specs=pl.BlockSpec((_EDGE_BLK, D), lambda i: (i, 0)),
        out_shape=jax.ShapeDtypeStruct((N_EDGES, D), F32),
    )(gd, gs, ea, we, bias)


def _stats_update(st_ref, y, i):
    s = jnp.sum(y, axis=0)[None, :]
    q = jnp.sum(y * y, axis=0)[None, :]
    upd = jnp.concatenate([s, q, jnp.zeros((6, D), F32)], axis=0)

    @pl.when(i == 0)
    def _():
        st_ref[...] = upd

    @pl.when(i > 0)
    def _():
        st_ref[...] = st_ref[...] + upd


def _add_stats_body(x_ref, aa_ref, ab_ref, y_ref, st_ref):
    i = pl.program_id(0)
    y = x_ref[...] + aa_ref[...] + ab_ref[...]
    y_ref[...] = y
    _stats_update(st_ref, y, i)


def _add_stats(x, agg_a, agg_b):
    return pl.pallas_call(
        _add_stats_body,
        grid=(_N_NODE_BLKS,),
        in_specs=[pl.BlockSpec((_NODE_BLK, D), lambda i: (i, 0))] * 3,
        out_specs=[pl.BlockSpec((_NODE_BLK, D), lambda i: (i, 0)),
                   pl.BlockSpec((8, D), lambda i: (0, 0))],
        out_shape=[jax.ShapeDtypeStruct((N_NODES, D), F32),
                   jax.ShapeDtypeStruct((8, D), F32)],
    )(x, agg_a, agg_b)


def _finalize_body(h_ref, aa_ref, ab_ref, b_ref, st_ref, pool_ref, cnt_ref):
    i = pl.program_id(0)
    y = h_ref[...] + aa_ref[...] + ab_ref[...]
    _stats_update(st_ref, y, i)
    lanes = lax.broadcasted_iota(jnp.int32, (_NODE_BLK, D), 1)
    oh = (b_ref[...] == lanes).astype(F32)
    pool_upd = lax.dot_general(oh, y, (((0,), (0,)), ((), ())),
                               preferred_element_type=F32)
    cnt_upd = jnp.concatenate(
        [jnp.sum(oh, axis=0)[None, :], jnp.zeros((7, D), F32)], axis=0)

    @pl.when(i == 0)
    def _():
        pool_ref[...] = pool_upd
        cnt_ref[...] = cnt_upd

    @pl.when(i > 0)
    def _():
        pool_ref[...] = pool_ref[...] + pool_upd
        cnt_ref[...] = cnt_ref[...] + cnt_upd


def _finalize(h, agg_a, agg_b, batch2d):
    return pl.pallas_call(
        _finalize_body,
        grid=(_N_NODE_BLKS,),
        in_specs=[pl.BlockSpec((_NODE_BLK, D), lambda i: (i, 0))] * 3 +
                 [pl.BlockSpec((_NODE_BLK, 1), lambda i: (i, 0))],
        out_specs=[pl.BlockSpec((8, D), lambda i: (0, 0)),
                   pl.BlockSpec((D, D), lambda i: (0, 0)),
                   pl.BlockSpec((8, D), lambda i: (0, 0))],
        out_shape=[jax.ShapeDtypeStruct((8, D), F32),
                   jax.ShapeDtypeStruct((D, D), F32),
                   jax.ShapeDtypeStruct((8, D), F32)],
    )(h, agg_a, agg_b, batch2d)


def _bn_affine(st, gamma, beta, eps=1e-5):
    mean = st[0] / N_NODES
    var = st[1] / N_NODES - mean * mean
    a = gamma * lax.rsqrt(var + eps)
    return a, beta - mean * a


def _pack_weights(wf, ws):
    wc = jnp.concatenate(
        [wf[:D], ws[:D], wf[D:2 * D], ws[D:2 * D]], axis=1)
    we = jnp.concatenate([wf[2 * D:], ws[2 * D:]], axis=1)
    return wc, we


def _pad8(row_list):
    rows = jnp.stack(row_list, axis=0)
    return jnp.concatenate(
        [rows, jnp.zeros((8 - rows.shape[0], rows.shape[1]), F32)], axis=0)


def kernel(x, edge_index, edge_attr, batch, Wf0, bf0, Ws0, bs0, g0, b0,
           Wf1, bf1, Ws1, bs1, g1, b1):
    src = edge_index[0]
    dst = edge_index[1]
    zeros_nd = jnp.zeros((N_PAD, D), F32)
    batch2d = batch.reshape(N_NODES, 1)

    wc0, we0 = _pack_weights(Wf0, Ws0)
    wc1, we1 = _pack_weights(Wf1, Ws1)
    bias0 = _pad8([jnp.concatenate([bf0, bs0])])
    bias1 = _pad8([jnp.concatenate([bf1, bs1])])
    ac_id = _pad8([jnp.ones((D,), F32), jnp.zeros((D,), F32)])
    sc_gather, sc_scatter = _sc_kernels()

    _, pd1, ps1 = _norm_matmul(x, ac_id, wc0)
    gd1, gs1 = sc_gather(pd1, ps1, src, dst)
    msg1 = _msg_kernel(gd1, gs1, edge_attr, we0, bias0)
    agg1a, agg1b = sc_scatter(msg1, dst, zeros_nd)
    y1, st1 = _add_stats(x, agg1a[:N_NODES], agg1b[:N_NODES])
    a1, c1 = _bn_affine(st1, g0, b0)

    h1, pd2, ps2 = _norm_matmul(y1, _pad8([a1, c1]), wc1)
    gd2, gs2 = sc_gather(pd2, ps2, src, dst)
    msg2 = _msg_kernel(gd2, gs2, edge_attr, we1, bias1)
    agg2a, agg2b = sc_scatter(msg2, dst, zeros_nd)
    st2, pooled, cnt = _finalize(h1, agg2a[:N_NODES], agg2b[:N_NODES], batch2d)

    a2, c2 = _bn_affine(st2, g1, b1)
    counts = jnp.maximum(cnt[0, :N_GRAPHS], 1.0)
    pool_mean = pooled[:N_GRAPHS] / counts[:, None]
    return pool_mean * a2[None, :] + c2[None, :]


# speedup vs baseline: 3.0804x; 1.2829x over previous
import functools

import jax
import jax.numpy as jnp
from jax import lax
from jax.experimental import pallas as pl
from jax.experimental.pallas import tpu as pltpu
from jax.experimental.pallas import tpu_sc as plsc

N_NODES = 10000
N_EDGES = 320000
D = 128
D_EDGE = 16
N_GRAPHS = 64

NC = 2
NS = 16
NW = NC * NS
EDGES_PER_W = N_EDGES // NW
CHUNK = 80
NCHUNK = EDGES_PER_W // CHUNK
N_PAD = 10240
ROWS_PER_TILE = N_PAD // NS

F32 = jnp.float32


@functools.cache
def _sc_kernels():
    mesh = plsc.VectorSubcoreMesh(
        core_axis_name="c", subcore_axis_name="s",
        num_cores=NC, num_subcores=NS)

    @functools.partial(
        pl.kernel,
        out_type=[jax.ShapeDtypeStruct((N_EDGES, D), jnp.int32),
                  jax.ShapeDtypeStruct((N_EDGES, D), jnp.int32)],
        mesh=mesh,
        scratch_types=[pltpu.VMEM((CHUNK,), jnp.int32),
                       pltpu.VMEM((CHUNK,), jnp.int32),
                       pltpu.VMEM((CHUNK,), jnp.int32),
                       pltpu.VMEM((CHUNK,), jnp.int32),
                       pltpu.VMEM((CHUNK, D), jnp.int32),
                       pltpu.VMEM((CHUNK, D), jnp.int32),
                       pltpu.VMEM((CHUNK, D), jnp.int32),
                       pltpu.VMEM((CHUNK, D), jnp.int32),
                       pltpu.SemaphoreType.DMA,
                       pltpu.SemaphoreType.DMA,
                       pltpu.SemaphoreType.DMA,
                       pltpu.SemaphoreType.DMA],
    )
    def _sc_gather(pd_hbm, ps_hbm, src_hbm, dst_hbm, gd_out, gs_out,
                   ida_d, ida_s, idb_d, idb_s,
                   rowsa_d, rowsa_s, rowsb_d, rowsb_s,
                   sema_d, sema_s, semb_d, semb_s):
        c = lax.axis_index("c")
        s = lax.axis_index("s")
        w = c * NS + s
        first = w * EDGES_PER_W

        def load_idx(base, id_d, id_s):
            pltpu.sync_copy(dst_hbm.at[pl.ds(base, CHUNK)], id_d)
            pltpu.sync_copy(src_hbm.at[pl.ds(base, CHUNK)], id_s)

        def start(id_d, id_s, r_d, r_s, se_d, se_s):
            pltpu.async_copy(pd_hbm.at[id_d], r_d, se_d)
            pltpu.async_copy(ps_hbm.at[id_s], r_s, se_s)

        def drain(id_d, id_s, r_d, r_s, se_d, se_s):
            pltpu.make_async_copy(pd_hbm.at[id_d], r_d, se_d).wait()
            pltpu.make_async_copy(ps_hbm.at[id_s], r_s, se_s).wait()

        def store(base, r_d, r_s):
            pltpu.sync_copy(r_d, gd_out.at[pl.ds(base, CHUNK)])
            pltpu.sync_copy(r_s, gs_out.at[pl.ds(base, CHUNK)])

        load_idx(first, ida_d, ida_s)
        start(ida_d, ida_s, rowsa_d, rowsa_s, sema_d, sema_s)

        def body(i, carry):
            base_a = first + 2 * i * CHUNK
            base_b = base_a + CHUNK
            load_idx(base_b, idb_d, idb_s)
            start(idb_d, idb_s, rowsb_d, rowsb_s, semb_d, semb_s)
            drain(ida_d, ida_s, rowsa_d, rowsa_s, sema_d, sema_s)
            store(base_a, rowsa_d, rowsa_s)
            load_idx(base_b + CHUNK, ida_d, ida_s)
            start(ida_d, ida_s, rowsa_d, rowsa_s, sema_d, sema_s)
            drain(idb_d, idb_s, rowsb_d, rowsb_s, semb_d, semb_s)
            store(base_b, rowsb_d, rowsb_s)
            return carry

        lax.fori_loop(0, (NCHUNK - 1) // 2, body, 0)
        drain(ida_d, ida_s, rowsa_d, rowsa_s, sema_d, sema_s)
        store(first + (NCHUNK - 1) * CHUNK, rowsa_d, rowsa_s)

    @functools.partial(
        pl.kernel,
        out_type=[jax.ShapeDtypeStruct((N_PAD, D), F32),
                  jax.ShapeDtypeStruct((N_PAD, D), F32)],
        mesh=mesh,
        scratch_types=[pltpu.VMEM((CHUNK,), jnp.int32),
                       pltpu.VMEM((CHUNK,), jnp.int32),
                       pltpu.VMEM((CHUNK, D), F32),
                       pltpu.VMEM((CHUNK, D), F32),
                       pltpu.MemorySpace.VMEM_SHARED((N_PAD, D), F32),
                       pltpu.SemaphoreType.DMA,
                       pltpu.SemaphoreType.DMA],
    )
    def _sc_scatter(msg_hbm, dst_hbm, zeros_hbm, out_a, out_b,
                    idxa, idxb, msga, msgb, agg_sh, sema, semb):
        c = lax.axis_index("c")
        s = lax.axis_index("s")
        row0 = s * ROWS_PER_TILE
        first = (c * NS + s) * EDGES_PER_W
        pltpu.sync_copy(zeros_hbm.at[pl.ds(row0, ROWS_PER_TILE)],
                        agg_sh.at[pl.ds(row0, ROWS_PER_TILE)])
        plsc.subcore_barrier()

        def load(base, idx_v, msg_v):
            pltpu.sync_copy(dst_hbm.at[pl.ds(base, CHUNK)], idx_v)
            pltpu.sync_copy(msg_hbm.at[pl.ds(base, CHUNK)], msg_v)

        load(first, idxa, msga)

        def body(i, carry):
            base_b = first + (2 * i + 1) * CHUNK
            pltpu.async_copy(msga, agg_sh.at[idxa], sema, add=True)
            load(base_b, idxb, msgb)
            pltpu.make_async_copy(msga, agg_sh.at[idxa], sema).wait()
            pltpu.async_copy(msgb, agg_sh.at[idxb], semb, add=True)
            load(base_b + CHUNK, idxa, msga)
            pltpu.make_async_copy(msgb, agg_sh.at[idxb], semb).wait()
            return carry

        lax.fori_loop(0, (NCHUNK - 1) // 2, body, 0)
        pltpu.async_copy(msga, agg_sh.at[idxa], sema, add=True)
        pltpu.make_async_copy(msga, agg_sh.at[idxa], sema).wait()
        plsc.subcore_barrier()

        @pl.when(c == 0)
        def _():
            pltpu.sync_copy(agg_sh.at[pl.ds(row0, ROWS_PER_TILE)],
                            out_a.at[pl.ds(row0, ROWS_PER_TILE)])

        @pl.when(c == 1)
        def _():
            pltpu.sync_copy(agg_sh.at[pl.ds(row0, ROWS_PER_TILE)],
                            out_b.at[pl.ds(row0, ROWS_PER_TILE)])

    return _sc_gather, _sc_scatter


_NODE_BLK = 1000
_N_NODE_BLKS = N_NODES // _NODE_BLK
_EDGE_BLK = 2000
_N_EDGE_BLKS = N_EDGES // _EDGE_BLK


def _bf16_bits(x):
    u = jax.lax.bitcast_convert_type(x, jnp.int32)
    r = u + jnp.int32(0x7FFF) + ((u >> 16) & jnp.int32(1))
    return jax.lax.shift_right_logical(r, 16)


def _pack_pair(f, s):
    return _bf16_bits(f) | jax.lax.shift_left(_bf16_bits(s), jnp.int32(16))


def _norm_matmul_body(y_ref, ac_ref, w_ref, h_ref, pd_ref, ps_ref):
    a = ac_ref[0:1, :]
    cc = ac_ref[1:2, :]
    h = y_ref[...] * a + cc
    h_ref[...] = h
    p = jnp.dot(h, w_ref[...], preferred_element_type=F32)
    pd_ref[...] = _pack_pair(p[:, :D], p[:, D:2 * D])
    ps_ref[...] = _pack_pair(p[:, 2 * D:3 * D], p[:, 3 * D:])


def _norm_matmul(y, ac, wc):
    return pl.pallas_call(
        _norm_matmul_body,
        grid=(_N_NODE_BLKS,),
        in_specs=[pl.BlockSpec((_NODE_BLK, D), lambda i: (i, 0)),
                  pl.BlockSpec((8, D), lambda i: (0, 0)),
                  pl.BlockSpec((D, 4 * D), lambda i: (0, 0))],
        out_specs=[pl.BlockSpec((_NODE_BLK, D), lambda i: (i, 0)),
                   pl.BlockSpec((_NODE_BLK, D), lambda i: (i, 0)),
                   pl.BlockSpec((_NODE_BLK, D), lambda i: (i, 0))],
        out_shape=[jax.ShapeDtypeStruct((N_NODES, D), F32),
                   jax.ShapeDtypeStruct((N_NODES, D), jnp.int32),
                   jax.ShapeDtypeStruct((N_NODES, D), jnp.int32)],
    )(y, ac, wc)


def _unpack_pair(w):
    f = jax.lax.bitcast_convert_type(jax.lax.shift_left(w, jnp.int32(16)), F32)
    s = jax.lax.bitcast_convert_type(
        jax.lax.shift_left(w >> 16, jnp.int32(16)), F32)
    return f, s


def _msg_body(gd_ref, gs_ref, ea_ref, we_ref, bias_ref, msg_ref):
    e2 = jnp.dot(ea_ref[...], we_ref[...], preferred_element_type=F32)
    e2 = e2 + bias_ref[0:1, :]
    fd, sd = _unpack_pair(gd_ref[...])
    fs, ss = _unpack_pair(gs_ref[...])
    zf = fd + fs + e2[:, :D]
    zs = sd + ss + e2[:, D:]
    msg_ref[...] = jax.nn.sigmoid(zf) * jax.nn.softplus(zs)


def _msg_kernel(gd, gs, ea, we, bias):
    return pl.pallas_call(
        _msg_body,
        grid=(_N_EDGE_BLKS,),
        in_specs=[pl.BlockSpec((_EDGE_BLK, D), lambda i: (i, 0)),
                  pl.BlockSpec((_EDGE_BLK, D), lambda i: (i, 0)),
                  pl.BlockSpec((_EDGE_BLK, D_EDGE), lambda i: (i, 0)),
                  pl.BlockSpec((D_EDGE, 2 * D), lambda i: (0, 0)),
                  pl.BlockSpec((8, 2 * D), lambda i: (0, 0))],
        out_specs=pl.BlockSpec((_EDGE_BLK, D), lambda i: (i, 0)),
        out_shape=jax.ShapeDtypeStruct((N_EDGES, D), F32),
    )(gd, gs, ea, we, bias)


def _stats_update(st_ref, y, i):
    s = jnp.sum(y, axis=0)[None, :]
    q = jnp.sum(y * y, axis=0)[None, :]
    upd = jnp.concatenate([s, q, jnp.zeros((6, D), F32)], axis=0)

    @pl.when(i == 0)
    def _():
        st_ref[...] = upd

    @pl.when(i > 0)
    def _():
        st_ref[...] = st_ref[...] + upd


def _add_stats_body(x_ref, aa_ref, ab_ref, y_ref, st_ref):
    i = pl.program_id(0)
    y = x_ref[...] + aa_ref[...] + ab_ref[...]
    y_ref[...] = y
    _stats_update(st_ref, y, i)


def _add_stats(x, agg_a, agg_b):
    return pl.pallas_call(
        _add_stats_body,
        grid=(_N_NODE_BLKS,),
        in_specs=[pl.BlockSpec((_NODE_BLK, D), lambda i: (i, 0))] * 3,
        out_specs=[pl.BlockSpec((_NODE_BLK, D), lambda i: (i, 0)),
                   pl.BlockSpec((8, D), lambda i: (0, 0))],
        out_shape=[jax.ShapeDtypeStruct((N_NODES, D), F32),
                   jax.ShapeDtypeStruct((8, D), F32)],
    )(x, agg_a, agg_b)


def _finalize_body(h_ref, aa_ref, ab_ref, b_ref, st_ref, pool_ref, cnt_ref):
    i = pl.program_id(0)
    y = h_ref[...] + aa_ref[...] + ab_ref[...]
    _stats_update(st_ref, y, i)
    lanes = lax.broadcasted_iota(jnp.int32, (_NODE_BLK, D), 1)
    oh = (b_ref[...] == lanes).astype(F32)
    pool_upd = lax.dot_general(oh, y, (((0,), (0,)), ((), ())),
                               preferred_element_type=F32)
    cnt_upd = jnp.concatenate(
        [jnp.sum(oh, axis=0)[None, :], jnp.zeros((7, D), F32)], axis=0)

    @pl.when(i == 0)
    def _():
        pool_ref[...] = pool_upd
        cnt_ref[...] = cnt_upd

    @pl.when(i > 0)
    def _():
        pool_ref[...] = pool_ref[...] + pool_upd
        cnt_ref[...] = cnt_ref[...] + cnt_upd


def _finalize(h, agg_a, agg_b, batch2d):
    return pl.pallas_call(
        _finalize_body,
        grid=(_N_NODE_BLKS,),
        in_specs=[pl.BlockSpec((_NODE_BLK, D), lambda i: (i, 0))] * 3 +
                 [pl.BlockSpec((_NODE_BLK, 1), lambda i: (i, 0))],
        out_specs=[pl.BlockSpec((8, D), lambda i: (0, 0)),
                   pl.BlockSpec((D, D), lambda i: (0, 0)),
                   pl.BlockSpec((8, D), lambda i: (0, 0))],
        out_shape=[jax.ShapeDtypeStruct((8, D), F32),
                   jax.ShapeDtypeStruct((D, D), F32),
                   jax.ShapeDtypeStruct((8, D), F32)],
    )(h, agg_a, agg_b, batch2d)


def _bn_affine(st, gamma, beta, eps=1e-5):
    mean = st[0] / N_NODES
    var = st[1] / N_NODES - mean * mean
    a = gamma * lax.rsqrt(var + eps)
    return a, beta - mean * a


def _pack_weights(wf, ws):
    wc = jnp.concatenate(
        [wf[:D], ws[:D], wf[D:2 * D], ws[D:2 * D]], axis=1)
    we = jnp.concatenate([wf[2 * D:], ws[2 * D:]], axis=1)
    return wc, we


def _pad8(row_list):
    rows = jnp.stack(row_list, axis=0)
    return jnp.concatenate(
        [rows, jnp.zeros((8 - rows.shape[0], rows.shape[1]), F32)], axis=0)


def kernel(x, edge_index, edge_attr, batch, Wf0, bf0, Ws0, bs0, g0, b0,
           Wf1, bf1, Ws1, bs1, g1, b1):
    src = edge_index[0]
    dst = edge_index[1]
    zeros_nd = jnp.zeros((N_PAD, D), F32)
    batch2d = batch.reshape(N_NODES, 1)

    wc0, we0 = _pack_weights(Wf0, Ws0)
    wc1, we1 = _pack_weights(Wf1, Ws1)
    bias0 = _pad8([jnp.concatenate([bf0, bs0])])
    bias1 = _pad8([jnp.concatenate([bf1, bs1])])
    ac_id = _pad8([jnp.ones((D,), F32), jnp.zeros((D,), F32)])
    sc_gather, sc_scatter = _sc_kernels()

    _, pd1, ps1 = _norm_matmul(x, ac_id, wc0)
    gd1, gs1 = sc_gather(pd1, ps1, src, dst)
    msg1 = _msg_kernel(gd1, gs1, edge_attr, we0, bias0)
    agg1a, agg1b = sc_scatter(msg1, dst, zeros_nd)
    y1, st1 = _add_stats(x, agg1a[:N_NODES], agg1b[:N_NODES])
    a1, c1 = _bn_affine(st1, g0, b0)

    h1, pd2, ps2 = _norm_matmul(y1, _pad8([a1, c1]), wc1)
    gd2, gs2 = sc_gather(pd2, ps2, src, dst)
    msg2 = _msg_kernel(gd2, gs2, edge_attr, we1, bias1)
    agg2a, agg2b = sc_scatter(msg2, dst, zeros_nd)
    st2, pooled, cnt = _finalize(h1, agg2a[:N_NODES], agg2b[:N_NODES], batch2d)

    a2, c2 = _bn_affine(st2, g1, b1)
    counts = jnp.maximum(cnt[0, :N_GRAPHS], 1.0)
    pool_mean = pooled[:N_GRAPHS] / counts[:, None]
    return pool_mean * a2[None, :] + c2[None, :]
